# baseline (speedup 1.0000x reference)
import functools
import math

import jax
import jax.numpy as jnp
from jax import lax
from jax.experimental import pallas as pl
from jax.experimental.pallas import tpu as pltpu
from jax.experimental.pallas import tpu_sc as plsc

_N = 1024
_V = 100000
_PAD = 0
_SMOOTH_VAL = 0.1 / (_V - 2)
_CONF = 0.9
_K = (_V - 2) * _SMOOTH_VAL * math.log(_SMOOTH_VAL) + _CONF * math.log(_CONF)

_NC, _NS, _L = 2, 16, 16
_NW = _NC * _NS
_RPW = _N // _NW
_NVEC = _RPW // _L


def _sc_body(t_hbm, out_hbm, t_v, acc_v):
    wid = lax.axis_index("s") * _NC + lax.axis_index("c")
    base = wid * _RPW
    pltpu.sync_copy(t_hbm.at[pl.ds(base, _RPW)], t_v)
    acc = jnp.zeros((_L,), jnp.float32)
    for j in range(_NVEC):
        t16 = t_v[pl.ds(j * _L, _L)]
        acc = acc + jnp.where(t16 != _PAD, _K, 0.0)
    acc_v[...] = acc
    pltpu.sync_copy(acc_v, out_hbm.at[wid])


@functools.cache
def _sc_call():
    return functools.partial(
        pl.kernel,
        out_type=jax.ShapeDtypeStruct((_NW, _L), jnp.float32),
        mesh=plsc.VectorSubcoreMesh(core_axis_name="c", subcore_axis_name="s"),
        scratch_types=[
            pltpu.VMEM((_RPW,), jnp.int32),
            pltpu.VMEM((_L,), jnp.float32),
        ],
    )(_sc_body)


_BR = 4096
_GRID = -(-_V // _BR)


def _tc_body(t_ref, tc_ref, x_ref, out_ref):
    j = pl.program_id(0)
    xb = x_ref[...]
    w = jnp.where(tc_ref[...] != _PAD, 1.0, 0.0)

    @pl.when(j == 0)
    def _():
        row0 = lax.dot_general(xb[0:1, :], w, (((1,), (0,)), ((), ())),
                               preferred_element_type=jnp.float32)
        out_ref[...] = (_SMOOTH_VAL * row0[0, 0]).reshape(1, 1)

    vcol = j * _BR + lax.broadcasted_iota(jnp.int32, (_BR, _N), 0)
    y = xb * jnp.where(vcol == t_ref[...], -_CONF, -_SMOOTH_VAL)
    vec = lax.dot_general(y.astype(jnp.bfloat16), w.astype(jnp.bfloat16),
                          (((1,), (0,)), ((), ())),
                          preferred_element_type=jnp.float32)
    vrow = j * _BR + lax.broadcasted_iota(jnp.int32, (_BR, 1), 0)
    out_ref[...] += jnp.sum(jnp.where(vrow < _V, vec, 0.0)).reshape(1, 1)


_tc_call = pl.pallas_call(
    _tc_body,
    grid=(_GRID,),
    in_specs=[
        pl.BlockSpec((1, _N), lambda j: (0, 0)),
        pl.BlockSpec((_N, 1), lambda j: (0, 0)),
        pl.BlockSpec((_BR, _N), lambda j: (j, 0)),
    ],
    out_specs=pl.BlockSpec((1, 1), lambda j: (0, 0)),
    out_shape=jax.ShapeDtypeStruct((1, 1), jnp.float32),
)


def kernel(x, target):
    assert x.shape == (_N, _V)
    t32 = target.astype(jnp.int32)
    k_partials = _sc_call()(t32)
    tc_out = _tc_call(t32.reshape(1, _N), t32.reshape(_N, 1),
                      jnp.swapaxes(x, 0, 1))
    return tc_out[0, 0] + jnp.sum(k_partials)


# speedup vs baseline: 1.0228x; 1.0228x over previous
import functools
import math

import jax
import jax.numpy as jnp
from jax import lax
from jax.experimental import pallas as pl
from jax.experimental.pallas import tpu as pltpu
from jax.experimental.pallas import tpu_sc as plsc

_N = 1024
_V = 100000
_PAD = 0
_SMOOTH_VAL = 0.1 / (_V - 2)
_CONF = 0.9
_K = (_V - 2) * _SMOOTH_VAL * math.log(_SMOOTH_VAL) + _CONF * math.log(_CONF)

_NC, _NS, _L = 2, 16, 16
_NW = _NC * _NS
_RPW = _N // _NW
_NVEC = _RPW // _L


def _sc_body(t_hbm, out_hbm, t_v, acc_v):
    wid = lax.axis_index("s") * _NC + lax.axis_index("c")
    base = wid * _RPW
    pltpu.sync_copy(t_hbm.at[pl.ds(base, _RPW)], t_v)
    acc = jnp.zeros((_L,), jnp.float32)
    for j in range(_NVEC):
        t16 = t_v[pl.ds(j * _L, _L)]
        acc = acc + jnp.where(t16 != _PAD, _K, 0.0)
    acc_v[...] = acc
    pltpu.sync_copy(acc_v, out_hbm.at[wid])


@functools.cache
def _sc_call():
    return functools.partial(
        pl.kernel,
        out_type=jax.ShapeDtypeStruct((_NW, _L), jnp.float32),
        mesh=plsc.VectorSubcoreMesh(core_axis_name="c", subcore_axis_name="s"),
        scratch_types=[
            pltpu.VMEM((_RPW,), jnp.int32),
            pltpu.VMEM((_L,), jnp.float32),
        ],
    )(_sc_body)


_BR = 4000
_GRID = -(-_V // _BR)


def _tc_body(t_ref, tc_ref, x_ref, out_ref):
    j = pl.program_id(0)
    xb = x_ref[...]
    w = jnp.where(tc_ref[...] != _PAD, 1.0, 0.0)

    @pl.when(j == 0)
    def _():
        row0 = lax.dot_general(xb[0:1, :], w, (((1,), (0,)), ((), ())),
                               preferred_element_type=jnp.float32)
        out_ref[...] = (_SMOOTH_VAL * row0[0, 0]).reshape(1, 1)

    vcol = j * _BR + lax.broadcasted_iota(jnp.int32, (_BR, _N), 0)
    y = xb * jnp.where(vcol == t_ref[...], -_CONF, -_SMOOTH_VAL)
    vec = lax.dot_general(y, w, (((1,), (0,)), ((), ())),
                          preferred_element_type=jnp.float32)
    vrow = j * _BR + lax.broadcasted_iota(jnp.int32, (_BR, 1), 0)
    out_ref[...] += jnp.sum(jnp.where(vrow < _V, vec, 0.0)).reshape(1, 1)


_tc_call = pl.pallas_call(
    _tc_body,
    grid=(_GRID,),
    in_specs=[
        pl.BlockSpec((1, _N), lambda j: (0, 0)),
        pl.BlockSpec((_N, 1), lambda j: (0, 0)),
        pl.BlockSpec((_BR, _N), lambda j: (j, 0)),
    ],
    out_specs=pl.BlockSpec((1, 1), lambda j: (0, 0)),
    out_shape=jax.ShapeDtypeStruct((1, 1), jnp.float32),
)


def kernel(x, target):
    assert x.shape == (_N, _V)
    t32 = target.astype(jnp.int32)
    k_partials = _sc_call()(t32)
    tc_out = _tc_call(t32.reshape(1, _N), t32.reshape(_N, 1),
                      jnp.swapaxes(x, 0, 1))
    return tc_out[0, 0] + jnp.sum(k_partials)
